# baseline (speedup 1.0000x reference)
import functools

import jax
import jax.numpy as jnp
from jax.experimental import pallas as pl
from jax.experimental.pallas import tpu as pltpu

N = 8192
BLOCK_R = 512


def _nn_kernel(a_ref, bt_ref, out_ref, s_ref, c_ref):
    step = pl.program_id(0)

    @pl.when(step == 0)
    def _init():
        s_ref[0] = 0.0
        c_ref[0] = 0.0

    a = a_ref[...]
    bt = bt_ref[...]
    a2 = jnp.sum(a * a, axis=1, keepdims=True)
    b2 = jnp.sum(bt * bt, axis=0, keepdims=True)
    ab = jnp.dot(a, bt, preferred_element_type=jnp.float32)

    d2 = jnp.maximum(a2 + b2 - 2.0 * ab, 0.0)
    dist = jnp.min(d2, axis=1, keepdims=True)

    mask = dist <= 2.0
    s_ref[0] += jnp.sum(jnp.where(mask, dist, 0.0))
    c_ref[0] += jnp.sum(mask.astype(jnp.float32))

    @pl.when(step == pl.num_programs(0) - 1)
    def _fin():
        out_ref[...] = jnp.reshape(s_ref[0] / jnp.maximum(c_ref[0], 1.0), (1, 1))


@jax.jit
def _nn(a, bt):
    out = pl.pallas_call(
        _nn_kernel,
        grid=(N // BLOCK_R,),
        in_specs=[
            pl.BlockSpec((BLOCK_R, 8), lambda i: (i, 0)),
            pl.BlockSpec((8, N), lambda i: (0, 0)),
        ],
        out_specs=pl.BlockSpec((1, 1), lambda i: (0, 0)),
        out_shape=jax.ShapeDtypeStruct((1, 1), jnp.float32),
        scratch_shapes=[
            pltpu.SMEM((1,), jnp.float32),
            pltpu.SMEM((1,), jnp.float32),
        ],
    )(a, bt)
    return out[0, 0]


def kernel(input0, input1):
    a = jnp.zeros((N, 8), jnp.float32).at[:, :3].set(input0)
    bt = jnp.zeros((8, N), jnp.float32).at[:3, :].set(input1.T)
    return _nn(a, bt)


# speedup vs baseline: 1.2609x; 1.2609x over previous
import functools

import jax
import jax.numpy as jnp
from jax.experimental import pallas as pl
from jax.experimental.pallas import tpu as pltpu

N = 8192
BLOCK_R = 512


def _nn_kernel(a_ref, btn2_ref, out_ref, s_ref, c_ref):
    step = pl.program_id(0)

    @pl.when(step == 0)
    def _init():
        s_ref[0] = 0.0
        c_ref[0] = 0.0

    a = a_ref[...]
    a2 = jnp.sum(a * a, axis=1, keepdims=True)
    btn2 = btn2_ref[...]
    b2 = 0.25 * jnp.sum(btn2 * btn2, axis=0, keepdims=True)
    nab2 = jnp.dot(a, btn2, preferred_element_type=jnp.float32)
    m = jnp.min(nab2 + b2, axis=1, keepdims=True)
    dist = jnp.maximum(a2 + m, 0.0)

    mask = dist <= 2.0
    s_ref[0] += jnp.sum(jnp.where(mask, dist, 0.0))
    c_ref[0] += jnp.sum(mask.astype(jnp.float32))

    @pl.when(step == pl.num_programs(0) - 1)
    def _fin():
        out_ref[...] = jnp.reshape(s_ref[0] / jnp.maximum(c_ref[0], 1.0), (1, 1))


@jax.jit
def _nn(a, btn2):
    out = pl.pallas_call(
        _nn_kernel,
        grid=(N // BLOCK_R,),
        in_specs=[
            pl.BlockSpec((BLOCK_R, 8), lambda i: (i, 0)),
            pl.BlockSpec((8, N), lambda i: (0, 0)),
        ],
        out_specs=pl.BlockSpec((1, 1), lambda i: (0, 0)),
        out_shape=jax.ShapeDtypeStruct((1, 1), jnp.float32),
        scratch_shapes=[
            pltpu.SMEM((1,), jnp.float32),
            pltpu.SMEM((1,), jnp.float32),
        ],
    )(a, btn2)
    return out[0, 0]


def kernel(input0, input1):
    a = jnp.zeros((N, 8), jnp.float32).at[:, :3].set(input0)
    btn2 = jnp.zeros((8, N), jnp.float32).at[:3, :].set(-2.0 * input1.T)
    return _nn(a, btn2)


# speedup vs baseline: 1.3268x; 1.0523x over previous
import functools

import jax
import jax.numpy as jnp
from jax.experimental import pallas as pl
from jax.experimental.pallas import tpu as pltpu

N = 8192
BLOCK_C = 512


def _nn_kernel(bn2_ref, at_ref, out_ref, acc_ref):
    step = pl.program_id(0)

    bn2 = bn2_ref[...]
    at = at_ref[...]
    b2 = 0.25 * jnp.sum(bn2 * bn2, axis=1, keepdims=True)
    ut = jnp.dot(bn2, at, preferred_element_type=jnp.float32)
    u = ut + b2
    m = u
    size = BLOCK_C
    while size > 8:
        half = size // 2
        m = jnp.minimum(m[0:half, :], m[half:size, :])
        size = half

    @pl.when(step == 0)
    def _init():
        acc_ref[...] = m

    @pl.when(step != 0)
    def _acc():
        acc_ref[...] = jnp.minimum(acc_ref[...], m)

    @pl.when(step == pl.num_programs(0) - 1)
    def _fin():
        a2 = jnp.sum(at * at, axis=0, keepdims=True)
        mfull = jnp.min(acc_ref[...], axis=0, keepdims=True)
        dist = jnp.maximum(a2 + mfull, 0.0)
        mask = dist <= 2.0
        s = jnp.sum(jnp.where(mask, dist, 0.0))
        c = jnp.sum(mask.astype(jnp.float32))
        out_ref[...] = jnp.reshape(s / jnp.maximum(c, 1.0), (1, 1))


@jax.jit
def _nn(bn2, at):
    out = pl.pallas_call(
        _nn_kernel,
        grid=(N // BLOCK_C,),
        in_specs=[
            pl.BlockSpec((BLOCK_C, 8), lambda i: (i, 0)),
            pl.BlockSpec((8, N), lambda i: (0, 0)),
        ],
        out_specs=pl.BlockSpec((1, 1), lambda i: (0, 0)),
        out_shape=jax.ShapeDtypeStruct((1, 1), jnp.float32),
        scratch_shapes=[
            pltpu.VMEM((8, N), jnp.float32),
        ],
    )(bn2, at)
    return out[0, 0]


def kernel(input0, input1):
    bn2 = jnp.zeros((N, 8), jnp.float32).at[:, :3].set(-2.0 * input1)
    at = jnp.zeros((8, N), jnp.float32).at[:3, :].set(input0.T)
    return _nn(bn2, at)
